# baseline (speedup 1.0000x reference)
import functools

import jax
import jax.numpy as jnp
from jax import lax
from jax.experimental import pallas as pl
from jax.experimental.pallas import tpu as pltpu
from jax.experimental.pallas import tpu_sc as plsc

L = 50
B = 16384
D = 32
V_IN = 1000000
V_OUT = 1000

NC = 2
NS = 16
NW = NC * NS
BPW = B // NW


def _sc_embed_sum_body(obs_hbm, table_hbm, out_hbm, idx_v, acc_v, buf0, buf1,
                       sem_a, sem0, sem1):
  wid = lax.axis_index("s") * NC + lax.axis_index("c")
  base = wid * BPW
  pltpu.sync_copy(obs_hbm.at[:, pl.ds(base, BPW)], idx_v)

  def xf_body(i, _):
    sl = (i // (BPW // 16), pl.ds((i % (BPW // 16)) * 16, 16))
    v = idx_v[sl]
    s = (lax.shift_left(lax.shift_right_logical(v, 13), 13)
         | lax.shift_left(v & 2047, 2)
         | (lax.shift_right_logical(v, 11) & 3))
    idx_v[sl] = s
    return 0

  lax.fori_loop(0, L * BPW // 16, xf_body, 0)

  cp_acc = pltpu.async_copy(table_hbm.at[idx_v.at[0]], acc_v, sem_a)
  bufs = [buf0, buf1]
  sems = [sem0, sem1]
  cps = [None, None]
  cps[1] = pltpu.async_copy(table_hbm.at[idx_v.at[1]], bufs[1], sems[1])
  cp_acc.wait()

  for l in range(1, L):
    if l + 1 < L:
      cps[(l + 1) % 2] = pltpu.async_copy(
          table_hbm.at[idx_v.at[l + 1]], bufs[(l + 1) % 2], sems[(l + 1) % 2])
    cps[l % 2].wait()
    buf = bufs[l % 2]

    def add_body(i, _, buf=buf):
      r = i * 4
      for k in range(4):
        for h in range(2):
          plsc.addupdate(acc_v.at[r + k, pl.ds(h * 16, 16)],
                         buf[r + k, pl.ds(h * 16, 16)])
      return 0

    lax.fori_loop(0, BPW // 4, add_body, 0)

  pltpu.sync_copy(acc_v, out_hbm.at[pl.ds(base, BPW)])


@functools.cache
def _sc_embed_sum():
  return pl.kernel(
      _sc_embed_sum_body,
      out_type=jax.ShapeDtypeStruct((B, D), jnp.float32),
      mesh=plsc.VectorSubcoreMesh(
          core_axis_name="c", subcore_axis_name="s", num_cores=NC,
          num_subcores=NS),
      compiler_params=pltpu.CompilerParams(use_tc_tiling_on_sc=False),
      scratch_types=[
          pltpu.VMEM((L, BPW), jnp.int32),
          pltpu.VMEM((BPW, D), jnp.float32),
          pltpu.VMEM((BPW, D), jnp.float32),
          pltpu.VMEM((BPW, D), jnp.float32),
          pltpu.SemaphoreType.DMA,
          pltpu.SemaphoreType.DMA,
          pltpu.SemaphoreType.DMA,
      ],
  )


BM = 512

_KS0 = 0
_KS1 = 42
_KS2 = _KS0 ^ _KS1 ^ 0x1BD11BDA
_ROT = ((13, 15, 26, 6), (17, 29, 16, 24))
_F32_ONE_BITS = 0x3F800000


def _rotl(x, r):
  return lax.shift_left(x, jnp.int32(r)) | lax.shift_right_logical(
      x, jnp.int32(32 - r))


def _threefry_bits(x0, x1):
  ks = (jnp.int32(_KS0), jnp.int32(_KS1), jnp.int32(_KS2))
  x0 = x0 + ks[0]
  x1 = x1 + ks[1]
  for d in range(5):
    for r in _ROT[d % 2]:
      x0 = x0 + x1
      x1 = _rotl(x1, r)
      x1 = x0 ^ x1
    x0 = x0 + ks[(d + 1) % 3]
    x1 = x1 + ks[(d + 2) % 3] + jnp.int32(d + 1)
  return x0 ^ x1


_TCOLS = 8192


_NBLK = (V_IN + _TCOLS - 1) // _TCOLS
_QW = _TCOLS // 4
V_STORE = _NBLK * _TCOLS
_BN = B // 128


def _tc_prep_body(xt_ref, u_ref, out_ref):
  j = pl.program_id(0)
  x = xt_ref[...]
  xc = jnp.concatenate(
      [x[:, q * _QW:(q + 1) * _QW] for q in range(4)], axis=0)
  out_ref[...] = xc.T

  v = lax.broadcasted_iota(jnp.int32, (V_OUT, 128), 0)
  b = lax.broadcasted_iota(jnp.int32, (V_OUT, 128), 1) + j * 128
  x1 = b * V_OUT + v
  bits = _threefry_bits(jnp.zeros_like(x1), x1)
  fbits = lax.shift_right_logical(bits, 9) | jnp.int32(_F32_ONE_BITS)
  u_ref[...] = lax.bitcast_convert_type(fbits, jnp.float32) - 1.0


def _tc_prep(emb_t):
  clamp = lambda j: jnp.minimum(j, _NBLK - 1)
  return pl.pallas_call(
      _tc_prep_body,
      grid=(_BN,),
      in_specs=[pl.BlockSpec((D, _TCOLS), lambda j: (0, clamp(j)))],
      out_specs=[
          pl.BlockSpec((V_OUT, 128), lambda j: (0, j)),
          pl.BlockSpec((_TCOLS * D // 128, 128), lambda j: (clamp(j), 0)),
      ],
      out_shape=[
          jax.ShapeDtypeStruct((V_OUT, B), jnp.float32),
          jax.ShapeDtypeStruct((V_STORE * D // 128, 128), jnp.float32),
      ],
  )(emb_t)


def _tc_head_body(hid_ref, w_ref, b_ref, vw_ref, vb_ref, u_ref,
                  probs_ref, comms_ref, value_ref):
  h = jnp.maximum(hid_ref[...], 0.0)
  logits = lax.dot_general(
      w_ref[...], h, (((1,), (1,)), ((), ())),
      preferred_element_type=jnp.float32)
  p = jax.nn.sigmoid(logits + b_ref[...])
  probs_ref[...] = p
  comms_ref[...] = (u_ref[...] < p).astype(jnp.float32)
  value_ref[...] = (
      lax.dot_general(vw_ref[...], h, (((1,), (1,)), ((), ())),
                      preferred_element_type=jnp.float32)
      + vb_ref[...])


def _tc_head(hid, w, b2, vw, vb2, u):
  grid = (B // BM,)
  return pl.pallas_call(
      _tc_head_body,
      grid=grid,
      in_specs=[
          pl.BlockSpec((BM, D), lambda i: (i, 0)),
          pl.BlockSpec((V_OUT, D), lambda i: (0, 0)),
          pl.BlockSpec((V_OUT, 1), lambda i: (0, 0)),
          pl.BlockSpec((1, D), lambda i: (0, 0)),
          pl.BlockSpec((1, 1), lambda i: (0, 0)),
          pl.BlockSpec((V_OUT, BM), lambda i: (0, i)),
      ],
      out_specs=[
          pl.BlockSpec((V_OUT, BM), lambda i: (0, i)),
          pl.BlockSpec((V_OUT, BM), lambda i: (0, i)),
          pl.BlockSpec((1, BM), lambda i: (0, i)),
      ],
      out_shape=[
          jax.ShapeDtypeStruct((V_OUT, B), jnp.float32),
          jax.ShapeDtypeStruct((V_OUT, B), jnp.float32),
          jax.ShapeDtypeStruct((1, B), jnp.float32),
      ],
  )(hid, w, b2, vw, vb2, u)


def kernel(observation, emb_weight, out_w, out_b, val_w, val_b):
  obs = observation.astype(jnp.int32)
  u_t, emb_lin = _tc_prep(emb_weight.T)
  hid = _sc_embed_sum()(obs, emb_lin.reshape(V_STORE, D))
  probs_t, comms_t, value_t = _tc_head(
      hid,
      out_w,
      out_b.reshape(V_OUT, 1),
      val_w,
      val_b.reshape(1, 1),
      u_t,
  )
  return comms_t.T, probs_t.T, value_t.T


# speedup vs baseline: 1.1712x; 1.1712x over previous
import functools

import jax
import jax.numpy as jnp
from jax import lax
from jax.experimental import pallas as pl
from jax.experimental.pallas import tpu as pltpu
from jax.experimental.pallas import tpu_sc as plsc

L = 50
B = 16384
D = 32
V_IN = 1000000
V_OUT = 1000

NC = 2
NS = 16
NW = NC * NS
BPW = B // NW


def _sc_embed_sum_body(obs_hbm, table_hbm, out_hbm, idx_v, acc_v, buf0, buf1,
                       sem_a, sem0, sem1):
  wid = lax.axis_index("s") * NC + lax.axis_index("c")
  base = wid * BPW
  pltpu.sync_copy(obs_hbm.at[:, pl.ds(base, BPW)], idx_v)

  def xf_body(i, _):
    sl = (i // (BPW // 16), pl.ds((i % (BPW // 16)) * 16, 16))
    v = idx_v[sl]
    s = (lax.shift_left(lax.shift_right_logical(v, 12), 12)
         | lax.shift_left(v & 1023, 2)
         | (lax.shift_right_logical(v, 10) & 3))
    idx_v[sl] = s
    return 0

  lax.fori_loop(0, L * BPW // 16, xf_body, 0)

  cp_acc = pltpu.async_copy(table_hbm.at[idx_v.at[0]], acc_v, sem_a)
  bufs = [buf0, buf1]
  sems = [sem0, sem1]
  cps = [None, None]
  cps[1] = pltpu.async_copy(table_hbm.at[idx_v.at[1]], bufs[1], sems[1])
  cp_acc.wait()

  for l in range(1, L):
    if l + 1 < L:
      cps[(l + 1) % 2] = pltpu.async_copy(
          table_hbm.at[idx_v.at[l + 1]], bufs[(l + 1) % 2], sems[(l + 1) % 2])
    cps[l % 2].wait()
    buf = bufs[l % 2]

    def add_body(i, _, buf=buf):
      r = i * 4
      for k in range(4):
        for h in range(2):
          plsc.addupdate(acc_v.at[r + k, pl.ds(h * 16, 16)],
                         buf[r + k, pl.ds(h * 16, 16)])
      return 0

    lax.fori_loop(0, BPW // 4, add_body, 0)

  pltpu.sync_copy(acc_v, out_hbm.at[pl.ds(base, BPW)])


@functools.cache
def _sc_embed_sum():
  return pl.kernel(
      _sc_embed_sum_body,
      out_type=jax.ShapeDtypeStruct((B, D), jnp.float32),
      mesh=plsc.VectorSubcoreMesh(
          core_axis_name="c", subcore_axis_name="s", num_cores=NC,
          num_subcores=NS),
      compiler_params=pltpu.CompilerParams(use_tc_tiling_on_sc=False),
      scratch_types=[
          pltpu.VMEM((L, BPW), jnp.int32),
          pltpu.VMEM((BPW, D), jnp.float32),
          pltpu.VMEM((BPW, D), jnp.float32),
          pltpu.VMEM((BPW, D), jnp.float32),
          pltpu.SemaphoreType.DMA,
          pltpu.SemaphoreType.DMA,
          pltpu.SemaphoreType.DMA,
      ],
  )


BM = 512

_KS0 = 0
_KS1 = 42
_KS2 = _KS0 ^ _KS1 ^ 0x1BD11BDA
_ROT = ((13, 15, 26, 6), (17, 29, 16, 24))
_F32_ONE_BITS = 0x3F800000


def _rotl(x, r):
  return lax.shift_left(x, jnp.int32(r)) | lax.shift_right_logical(
      x, jnp.int32(32 - r))


def _threefry_bits(x0, x1):
  ks = (jnp.int32(_KS0), jnp.int32(_KS1), jnp.int32(_KS2))
  x0 = x0 + ks[0]
  x1 = x1 + ks[1]
  for d in range(5):
    for r in _ROT[d % 2]:
      x0 = x0 + x1
      x1 = _rotl(x1, r)
      x1 = x0 ^ x1
    x0 = x0 + ks[(d + 1) % 3]
    x1 = x1 + ks[(d + 2) % 3] + jnp.int32(d + 1)
  return x0 ^ x1


_TCOLS = 4096


_NBLK = (V_IN + _TCOLS - 1) // _TCOLS
_QW = _TCOLS // 4
V_STORE = _NBLK * _TCOLS


def _tc_detile_body(xt_ref, out_ref):
  x = xt_ref[...]
  xc = jnp.concatenate(
      [x[:, q * _QW:(q + 1) * _QW] for q in range(4)], axis=0)
  out_ref[...] = xc.T


def _tc_detile(emb_t):
  return pl.pallas_call(
      _tc_detile_body,
      grid=(_NBLK,),
      in_specs=[pl.BlockSpec((D, _TCOLS), lambda j: (0, j))],
      out_specs=pl.BlockSpec((_TCOLS * D // 128, 128), lambda j: (j, 0)),
      out_shape=jax.ShapeDtypeStruct((V_STORE * D // 128, 128), jnp.float32),
  )(emb_t)


def _tc_bits_body(u_ref):
  j = pl.program_id(0)
  v = lax.broadcasted_iota(jnp.int32, (V_OUT, BM), 0)
  b = lax.broadcasted_iota(jnp.int32, (V_OUT, BM), 1) + j * BM
  x1 = b * V_OUT + v
  bits = _threefry_bits(jnp.zeros_like(x1), x1)
  fbits = lax.shift_right_logical(bits, 9) | jnp.int32(_F32_ONE_BITS)
  u_ref[...] = lax.bitcast_convert_type(fbits, jnp.float32) - 1.0


def _tc_bits():
  return pl.pallas_call(
      _tc_bits_body,
      grid=(B // BM,),
      out_specs=pl.BlockSpec((V_OUT, BM), lambda j: (0, j)),
      out_shape=jax.ShapeDtypeStruct((V_OUT, B), jnp.float32),
  )()


def _tc_head_body(hid_ref, w_ref, b_ref, vw_ref, vb_ref, u_ref,
                  probs_ref, comms_ref, value_ref):
  h = jnp.maximum(hid_ref[...], 0.0)
  logits = lax.dot_general(
      w_ref[...], h, (((1,), (1,)), ((), ())),
      preferred_element_type=jnp.float32)
  p = jax.nn.sigmoid(logits + b_ref[...])
  probs_ref[...] = p
  comms_ref[...] = (u_ref[...] < p).astype(jnp.float32)
  value_ref[...] = (
      lax.dot_general(vw_ref[...], h, (((1,), (1,)), ((), ())),
                      preferred_element_type=jnp.float32)
      + vb_ref[...])


def _tc_head(hid, w, b2, vw, vb2, u):
  grid = (B // BM,)
  return pl.pallas_call(
      _tc_head_body,
      grid=grid,
      in_specs=[
          pl.BlockSpec((BM, D), lambda i: (i, 0)),
          pl.BlockSpec((V_OUT, D), lambda i: (0, 0)),
          pl.BlockSpec((V_OUT, 1), lambda i: (0, 0)),
          pl.BlockSpec((1, D), lambda i: (0, 0)),
          pl.BlockSpec((1, 1), lambda i: (0, 0)),
          pl.BlockSpec((V_OUT, BM), lambda i: (0, i)),
      ],
      out_specs=[
          pl.BlockSpec((V_OUT, BM), lambda i: (0, i)),
          pl.BlockSpec((V_OUT, BM), lambda i: (0, i)),
          pl.BlockSpec((1, BM), lambda i: (0, i)),
      ],
      out_shape=[
          jax.ShapeDtypeStruct((V_OUT, B), jnp.float32),
          jax.ShapeDtypeStruct((V_OUT, B), jnp.float32),
          jax.ShapeDtypeStruct((1, B), jnp.float32),
      ],
  )(hid, w, b2, vw, vb2, u)


def kernel(observation, emb_weight, out_w, out_b, val_w, val_b):
  obs = observation.astype(jnp.int32)
  emb_lin = _tc_detile(emb_weight.T)
  u_t = _tc_bits()
  hid = _sc_embed_sum()(obs, emb_lin.reshape(V_STORE, D))
  probs_t, comms_t, value_t = _tc_head(
      hid,
      out_w,
      out_b.reshape(V_OUT, 1),
      val_w,
      val_b.reshape(1, 1),
      u_t,
  )
  return comms_t.T, probs_t.T, value_t.T
